# baseline (speedup 1.0000x reference)
import functools

import jax
import jax.numpy as jnp
from jax.experimental import pallas as pl
from jax.experimental.pallas import tpu as pltpu

_NUM_EXPERTS = 8
_TOP_K = 2
_EMB = 1024
_MLP = 4096
_ROWS = 4096
_TILE = 128
_STEPS = _ROWS // _TILE + _NUM_EXPERTS - 1
_PROWS = _STEPS * _TILE
_MC = 1024
_NK = _MLP // _MC


def _k1_body(expert_ref, nact_ref, x_ref, w0_ref, w1_ref, act_ref,
             w0b_ref, w1b_ref):
    s = pl.program_id(1)
    new_expert = jnp.logical_or(
        s == 0, expert_ref[s] != expert_ref[jnp.maximum(s - 1, 0)])

    @pl.when(jnp.logical_and(new_expert, s < nact_ref[0]))
    def _():
        w0b_ref[...] = w0_ref[0].astype(jnp.bfloat16)
        w1b_ref[...] = w1_ref[0].astype(jnp.bfloat16)

    @pl.when(s < nact_ref[0])
    def _():
        x = x_ref[...]
        a0 = jax.lax.dot_general(x, w0b_ref[...], (((1,), (0,)), ((), ())),
                                 preferred_element_type=jnp.float32)
        a1 = jax.lax.dot_general(x, w1b_ref[...], (((1,), (0,)), ((), ())),
                                 preferred_element_type=jnp.float32)
        act_ref[...] = (a0 * jax.nn.sigmoid(a0) * a1).astype(jnp.bfloat16)


def _k2_body(expert_ref, nact_ref, act_ref, wo_ref, out_ref, wob_ref):
    s = pl.program_id(0)
    new_expert = jnp.logical_or(
        s == 0, expert_ref[s] != expert_ref[jnp.maximum(s - 1, 0)])

    @pl.when(jnp.logical_and(new_expert, s < nact_ref[0]))
    def _():
        wob_ref[...] = wo_ref[0].astype(jnp.bfloat16)

    @pl.when(s < nact_ref[0])
    def _():
        out_ref[...] = jax.lax.dot_general(
            act_ref[...], wob_ref[...], (((1,), (0,)), ((), ())),
            preferred_element_type=jnp.float32)


def _gmm(sorted_x, w0, w1, wo, step_expert, n_active):
    k1_spec = pltpu.PrefetchScalarGridSpec(
        num_scalar_prefetch=2,
        grid=(_NK, _STEPS),
        in_specs=[
            pl.BlockSpec((_TILE, _EMB), lambda k, s, e, n: (s, 0)),
            pl.BlockSpec((1, _EMB, _MC), lambda k, s, e, n: (e[s], 0, k)),
            pl.BlockSpec((1, _EMB, _MC), lambda k, s, e, n: (e[s], 0, k)),
        ],
        out_specs=pl.BlockSpec((_TILE, _MC), lambda k, s, e, n: (s, k)),
        scratch_shapes=[
            pltpu.VMEM((_EMB, _MC), jnp.bfloat16),
            pltpu.VMEM((_EMB, _MC), jnp.bfloat16),
        ],
    )
    act = pl.pallas_call(
        _k1_body,
        grid_spec=k1_spec,
        out_shape=jax.ShapeDtypeStruct((_PROWS, _MLP), jnp.bfloat16),
        compiler_params=pltpu.CompilerParams(
            vmem_limit_bytes=60 * 1024 * 1024),
    )(step_expert, n_active, sorted_x, w0, w1)

    k2_spec = pltpu.PrefetchScalarGridSpec(
        num_scalar_prefetch=2,
        grid=(_STEPS,),
        in_specs=[
            pl.BlockSpec((_TILE, _MLP), lambda s, e, n: (s, 0)),
            pl.BlockSpec((1, _MLP, _EMB), lambda s, e, n: (e[s], 0, 0)),
        ],
        out_specs=pl.BlockSpec((_TILE, _EMB), lambda s, e, n: (s, 0)),
        scratch_shapes=[pltpu.VMEM((_MLP, _EMB), jnp.bfloat16)],
    )
    return pl.pallas_call(
        _k2_body,
        grid_spec=k2_spec,
        out_shape=jax.ShapeDtypeStruct((_PROWS, _EMB), jnp.float32),
        compiler_params=pltpu.CompilerParams(
            vmem_limit_bytes=60 * 1024 * 1024),
    )(step_expert, n_active, act, wo)


@jax.jit
def kernel(inputs, w_gate, w0_kernel, w1_kernel, wo_kernel):
    x2d = inputs.reshape(-1, _EMB)
    logits = x2d @ w_gate
    weights, selected = jax.lax.top_k(logits, _TOP_K)
    weights = jax.nn.softmax(weights.astype(jnp.float32), axis=-1)
    flat = selected.ravel()
    group_sizes = jnp.bincount(flat, length=_NUM_EXPERTS)

    tiles_per = (group_sizes + _TILE - 1) // _TILE
    tile_starts = jnp.concatenate([jnp.zeros((1,), jnp.int32),
                                   jnp.cumsum(tiles_per).astype(jnp.int32)])
    padded_offs = tile_starts[:-1] * _TILE
    n_active = tile_starts[-1:].astype(jnp.int32)

    s_idx = jnp.arange(_STEPS, dtype=jnp.int32)
    step_expert = jnp.sum(
        (s_idx[:, None] >= tile_starts[None, 1:]).astype(jnp.int32), axis=1)
    step_expert = jnp.minimum(step_expert, _NUM_EXPERTS - 1)

    sort_idx = jnp.argsort(flat)
    ranks = jnp.arange(_ROWS, dtype=jnp.int32) - jnp.take(
        jnp.concatenate([jnp.zeros((1,), jnp.int32),
                         jnp.cumsum(group_sizes).astype(jnp.int32)])[:-1],
        jnp.sort(flat))
    pos_sorted = jnp.take(padded_offs, jnp.sort(flat)) + ranks
    sorted_x = jnp.zeros((_PROWS, _EMB), jnp.bfloat16)
    sorted_x = sorted_x.at[pos_sorted].set(
        jnp.take(x2d, sort_idx // _TOP_K, axis=0).astype(jnp.bfloat16))

    inter = _gmm(sorted_x, w0_kernel, w1_kernel, wo_kernel,
                 step_expert, n_active)

    pos = jnp.zeros((_ROWS,), jnp.int32).at[sort_idx].set(pos_sorted)
    gathered = jnp.take(inter, pos.reshape(-1, _TOP_K), axis=0)
    out = jnp.einsum("tke,tk->te", gathered, weights)
    return out.reshape(inputs.shape)


# speedup vs baseline: 1.1344x; 1.1344x over previous
import functools

import jax
import jax.numpy as jnp
from jax import lax
from jax.experimental import pallas as pl
from jax.experimental.pallas import tpu as pltpu
from jax.experimental.pallas import tpu_sc as plsc

_NUM_EXPERTS = 8
_TOP_K = 2
_EMB = 1024
_MLP = 4096
_ROWS = 4096
_TILE = 128
_STEPS = _ROWS // _TILE + _NUM_EXPERTS - 1
_PROWS = _STEPS * _TILE
_MC = 1024
_NK = _MLP // _MC


_NC = 2
_NW = 32
_TOKENS = 2048


def _sc_mesh():
    return plsc.VectorSubcoreMesh(core_axis_name="c", subcore_axis_name="s")


def _dispatch(x_bf, pos):
    ppc = _ROWS // _NW // 2

    @functools.partial(
        pl.kernel, mesh=_sc_mesh(),
        out_type=jax.ShapeDtypeStruct((_PROWS, _EMB), jnp.float32),
        scratch_types=[
            pltpu.VMEM((ppc,), jnp.int32),
            pltpu.VMEM((ppc,), jnp.int32),
            pltpu.VMEM((ppc, _EMB), jnp.float32),
            pltpu.SemaphoreType.DMA,
        ],
    )
    def k(x_hbm, pos_hbm, out_hbm, tok_v, pos_v, rows_v, sem):
        wid = lax.axis_index("s") * _NC + lax.axis_index("c")
        iota = lax.iota(jnp.int32, 16)
        for c in range(2):
            base = wid * 2 * ppc + c * ppc
            pltpu.sync_copy(pos_hbm.at[pl.ds(base, ppc)], pos_v)
            for j in range(ppc // 16):
                tok_v[pl.ds(j * 16, 16)] = (base + j * 16 + iota) >> 1
            pltpu.async_copy(x_hbm.at[tok_v], rows_v, sem).wait()
            pltpu.async_copy(rows_v, out_hbm.at[pos_v], sem).wait()

    return k(x_bf, pos)


def _combine(inter, pos, w_flat):
    tpw = _TOKENS // _NW
    _TC = 8

    @functools.partial(
        pl.kernel, mesh=_sc_mesh(),
        out_type=jax.ShapeDtypeStruct((_TOKENS, _EMB), jnp.float32),
        scratch_types=[
            pltpu.VMEM((2 * _TC,), jnp.int32),
            pltpu.VMEM((2 * _TC, 16), jnp.float32),
            pltpu.VMEM((2 * _TC, _EMB), jnp.float32),
            pltpu.VMEM((_TC, _EMB), jnp.float32),
            pltpu.SemaphoreType.DMA,
        ],
    )
    def k(inter_hbm, pos_hbm, w_hbm, out_hbm, pidx_v, w_v, rows_v, out_v, sem):
        wid = lax.axis_index("s") * _NC + lax.axis_index("c")

        def body(c, carry):
            tok0 = wid * tpw + c * _TC
            pb = tok0 * 2
            pltpu.sync_copy(pos_hbm.at[pl.ds(pb, 2 * _TC)], pidx_v)
            pltpu.sync_copy(w_hbm.at[pl.ds(pb, 2 * _TC)], w_v)
            pltpu.async_copy(inter_hbm.at[pidx_v], rows_v, sem).wait()
            for i in range(_TC):
                w0 = w_v[2 * i, :]
                w1 = w_v[2 * i + 1, :]
                for j in range(_EMB // 16):
                    r0 = rows_v[2 * i, pl.ds(j * 16, 16)]
                    r1 = rows_v[2 * i + 1, pl.ds(j * 16, 16)]
                    out_v[i, pl.ds(j * 16, 16)] = w0 * r0 + w1 * r1
            pltpu.sync_copy(out_v, out_hbm.at[pl.ds(tok0, _TC)])
            return carry

        lax.fori_loop(0, tpw // _TC, body, 0)

    w_exp = jnp.broadcast_to(w_flat[:, None], (_ROWS, 16))
    return k(inter, pos, w_exp)


def _k1_body(expert_ref, nact_ref, x_ref, w0_ref, w1_ref, act_ref,
             w0b_ref, w1b_ref):
    s = pl.program_id(1)
    new_expert = jnp.logical_or(
        s == 0, expert_ref[s] != expert_ref[jnp.maximum(s - 1, 0)])

    @pl.when(jnp.logical_and(new_expert, s < nact_ref[0]))
    def _():
        w0b_ref[...] = w0_ref[0].astype(jnp.bfloat16)
        w1b_ref[...] = w1_ref[0].astype(jnp.bfloat16)

    @pl.when(s < nact_ref[0])
    def _():
        x = x_ref[...].astype(jnp.bfloat16)
        a0 = jax.lax.dot_general(x, w0b_ref[...], (((1,), (0,)), ((), ())),
                                 preferred_element_type=jnp.float32)
        a1 = jax.lax.dot_general(x, w1b_ref[...], (((1,), (0,)), ((), ())),
                                 preferred_element_type=jnp.float32)
        act_ref[...] = (a0 * jax.nn.sigmoid(a0) * a1).astype(jnp.bfloat16)


def _k2_body(expert_ref, nact_ref, act_ref, wo_ref, out_ref, wob_ref):
    s = pl.program_id(0)
    new_expert = jnp.logical_or(
        s == 0, expert_ref[s] != expert_ref[jnp.maximum(s - 1, 0)])

    @pl.when(jnp.logical_and(new_expert, s < nact_ref[0]))
    def _():
        wob_ref[...] = wo_ref[0].astype(jnp.bfloat16)

    @pl.when(s < nact_ref[0])
    def _():
        out_ref[...] = jax.lax.dot_general(
            act_ref[...], wob_ref[...], (((1,), (0,)), ((), ())),
            preferred_element_type=jnp.float32)


def _gmm(sorted_x, w0, w1, wo, step_expert, n_active):
    k1_spec = pltpu.PrefetchScalarGridSpec(
        num_scalar_prefetch=2,
        grid=(_NK, _STEPS),
        in_specs=[
            pl.BlockSpec((_TILE, _EMB), lambda k, s, e, n: (s, 0)),
            pl.BlockSpec((1, _EMB, _MC), lambda k, s, e, n: (e[s], 0, k)),
            pl.BlockSpec((1, _EMB, _MC), lambda k, s, e, n: (e[s], 0, k)),
        ],
        out_specs=pl.BlockSpec((_TILE, _MC), lambda k, s, e, n: (s, k)),
        scratch_shapes=[
            pltpu.VMEM((_EMB, _MC), jnp.bfloat16),
            pltpu.VMEM((_EMB, _MC), jnp.bfloat16),
        ],
    )
    act = pl.pallas_call(
        _k1_body,
        grid_spec=k1_spec,
        out_shape=jax.ShapeDtypeStruct((_PROWS, _MLP), jnp.bfloat16),
        compiler_params=pltpu.CompilerParams(
            vmem_limit_bytes=60 * 1024 * 1024),
    )(step_expert, n_active, sorted_x, w0, w1)

    k2_spec = pltpu.PrefetchScalarGridSpec(
        num_scalar_prefetch=2,
        grid=(_STEPS,),
        in_specs=[
            pl.BlockSpec((_TILE, _MLP), lambda s, e, n: (s, 0)),
            pl.BlockSpec((1, _MLP, _EMB), lambda s, e, n: (e[s], 0, 0)),
        ],
        out_specs=pl.BlockSpec((_TILE, _EMB), lambda s, e, n: (s, 0)),
        scratch_shapes=[pltpu.VMEM((_MLP, _EMB), jnp.bfloat16)],
    )
    return pl.pallas_call(
        _k2_body,
        grid_spec=k2_spec,
        out_shape=jax.ShapeDtypeStruct((_PROWS, _EMB), jnp.float32),
        compiler_params=pltpu.CompilerParams(
            vmem_limit_bytes=60 * 1024 * 1024),
    )(step_expert, n_active, act, wo)


@jax.jit
def kernel(inputs, w_gate, w0_kernel, w1_kernel, wo_kernel):
    x2d = inputs.reshape(-1, _EMB)
    logits = x2d @ w_gate
    weights, selected = jax.lax.top_k(logits, _TOP_K)
    weights = jax.nn.softmax(weights.astype(jnp.float32), axis=-1)
    flat = selected.ravel()
    group_sizes = jnp.bincount(flat, length=_NUM_EXPERTS)

    tiles_per = (group_sizes + _TILE - 1) // _TILE
    tile_starts = jnp.concatenate([jnp.zeros((1,), jnp.int32),
                                   jnp.cumsum(tiles_per).astype(jnp.int32)])
    padded_offs = tile_starts[:-1] * _TILE
    n_active = tile_starts[-1:].astype(jnp.int32)

    s_idx = jnp.arange(_STEPS, dtype=jnp.int32)
    step_expert = jnp.sum(
        (s_idx[:, None] >= tile_starts[None, 1:]).astype(jnp.int32), axis=1)
    step_expert = jnp.minimum(step_expert, _NUM_EXPERTS - 1)

    sort_idx = jnp.argsort(flat)
    ranks = jnp.arange(_ROWS, dtype=jnp.int32) - jnp.take(
        jnp.concatenate([jnp.zeros((1,), jnp.int32),
                         jnp.cumsum(group_sizes).astype(jnp.int32)])[:-1],
        jnp.sort(flat))
    pos_sorted = jnp.take(padded_offs, jnp.sort(flat)) + ranks
    pos = jnp.zeros((_ROWS,), jnp.int32).at[sort_idx].set(pos_sorted)

    sorted_x = _dispatch(x2d, pos)
    inter = _gmm(sorted_x, w0_kernel, w1_kernel, wo_kernel,
                 step_expert, n_active)
    out = _combine(inter, pos, weights.reshape(-1))
    return out.reshape(inputs.shape)
